# Initial kernel scaffold; baseline (speedup 1.0000x reference)
import jax
import jax.numpy as jnp
from jax.experimental import pallas as pl


def kernel(x_nodes, e_feat, W_self0, W_neigh0, b0, g0, beta0, W_self1, W_neigh1, b1, g1, beta1, Wm1, bm1, Wm2, bm2, edge_index0, edge_index1, pair_edges):
    raise NotImplementedError("write your pallas kernel here")


# speedup vs baseline: 13.1078x; 13.1078x over previous
import functools

import jax
import jax.numpy as jnp
from jax import lax
from jax.experimental import pallas as pl
from jax.experimental.pallas import tpu as pltpu
from jax.experimental.pallas import tpu_sc as plsc

NCSC = 2
NSUB = 16
NW = NCSC * NSUB
B = 128
D = 128


def _mesh():
    return plsc.VectorSubcoreMesh(
        core_axis_name="c", subcore_axis_name="s",
        num_cores=NCSC, num_subcores=NSUB)


def _make_segsum(n_table, e_pad, n_dst_pad):
    nb = e_pad // (B * NW)
    assert nb % 2 == 0 and nb * B * NW == e_pad
    rpt = n_dst_pad // NSUB
    assert rpt % 128 == 0

    @functools.partial(
        pl.kernel,
        out_type=(jax.ShapeDtypeStruct((NCSC, n_dst_pad, D), jnp.float32),
                  jax.ShapeDtypeStruct((NCSC, n_dst_pad), jnp.float32)),
        mesh=_mesh(),
        scratch_types=(
            pltpu.VMEM((nb, B), jnp.int32),
            pltpu.VMEM((nb, B), jnp.int32),
            pltpu.VMEM((B, D), jnp.float32),
            pltpu.VMEM((B, D), jnp.float32),
            pltpu.VMEM((B,), jnp.float32),
            pltpu.VMEM((16, D), jnp.float32),
            pltpu.VMEM((rpt,), jnp.float32),
            pltpu.VMEM_SHARED((n_dst_pad, D), jnp.float32),
            pltpu.VMEM_SHARED((n_dst_pad,), jnp.float32),
            pltpu.SemaphoreType.DMA,
            pltpu.SemaphoreType.DMA,
        ),
    )
    def seg(table, srcm, dstm, acc_out, deg_out,
            src_v, dst_v, row_a, row_b, ones_v, zrow, zdeg,
            acc_sh, deg_sh, sem_a, sem_b):
        c = lax.axis_index("c")
        s = lax.axis_index("s")
        w = s * NCSC + c
        base_r = s * rpt

        zero16 = jnp.zeros((16,), jnp.float32)
        one16 = jnp.full((16,), 1.0, jnp.float32)
        for r in range(16):
            for k in range(D // 16):
                zrow[r, pl.ds(16 * k, 16)] = zero16
        for k in range(B // 16):
            ones_v[pl.ds(16 * k, 16)] = one16
        for k in range(rpt // 16):
            zdeg[pl.ds(16 * k, 16)] = zero16

        for k in range(rpt // 16):
            pltpu.sync_copy(zrow, acc_sh.at[pl.ds(base_r + 16 * k, 16)])
        pltpu.sync_copy(zdeg, deg_sh.at[pl.ds(base_r, rpt)])

        pltpu.sync_copy(srcm.at[pl.ds(w * nb, nb)], src_v)
        pltpu.sync_copy(dstm.at[pl.ds(w * nb, nb)], dst_v)
        plsc.subcore_barrier()

        pltpu.async_copy(table.at[src_v.at[0]], row_a, sem_a)
        nh = nb // 2

        def body(i, carry):
            j = 2 * i
            pltpu.async_copy(table.at[src_v.at[j + 1]], row_b, sem_b)
            pltpu.make_async_copy(table.at[src_v.at[0]], row_a, sem_a).wait()
            pltpu.sync_copy(row_a, acc_sh.at[dst_v.at[j]], add=True)
            pltpu.sync_copy(ones_v, deg_sh.at[dst_v.at[j]], add=True)

            @pl.when(i + 1 < nh)
            def _():
                pltpu.async_copy(table.at[src_v.at[j + 2]], row_a, sem_a)

            pltpu.make_async_copy(table.at[src_v.at[0]], row_b, sem_b).wait()
            pltpu.sync_copy(row_b, acc_sh.at[dst_v.at[j + 1]], add=True)
            pltpu.sync_copy(ones_v, deg_sh.at[dst_v.at[j + 1]], add=True)
            return carry

        lax.fori_loop(0, nh, body, 0)
        plsc.subcore_barrier()

        pltpu.sync_copy(acc_sh.at[pl.ds(base_r, rpt)],
                        acc_out.at[c].at[pl.ds(base_r, rpt)])
        pltpu.sync_copy(deg_sh.at[pl.ds(base_r, rpt)],
                        deg_out.at[c].at[pl.ds(base_r, rpt)])

    return seg


def _make_gather(n_table, n_idx):
    nb = n_idx // (B * NW)
    assert nb * B * NW == n_idx

    @functools.partial(
        pl.kernel,
        out_type=jax.ShapeDtypeStruct((n_idx, D), jnp.float32),
        mesh=_mesh(),
        scratch_types=(
            pltpu.VMEM((nb, B), jnp.int32),
            pltpu.VMEM((B, D), jnp.float32),
            pltpu.SemaphoreType.DMA,
        ),
    )
    def gat(table, idxm, out, idx_v, row_v, sem):
        c = lax.axis_index("c")
        s = lax.axis_index("s")
        w = s * NCSC + c
        pltpu.sync_copy(idxm.at[pl.ds(w * nb, nb)], idx_v)
        for j in range(nb):
            pltpu.async_copy(table.at[idx_v.at[j]], row_v, sem).wait()
            pltpu.sync_copy(row_v, out.at[pl.ds((w * nb + j) * B, B)])

    return gat


def _sage_post(acc, deg, h_prev, w_self, w_neigh, b, g, beta, n_dst):
    def body(acc_ref, deg_ref, h_ref, ws_ref, wn_ref, b_ref, g_ref,
             beta_ref, out_ref):
        agg = acc_ref[0, :n_dst, :] + acc_ref[1, :n_dst, :]
        dg = deg_ref[0, :n_dst] + deg_ref[1, :n_dst]
        mean = agg / jnp.maximum(dg, 1.0)[:, None]
        z = (jnp.dot(h_ref[:n_dst, :], ws_ref[...],
                     preferred_element_type=jnp.float32)
             + jnp.dot(mean, wn_ref[...],
                       preferred_element_type=jnp.float32)
             + b_ref[...])
        mu = jnp.mean(z, axis=0)
        var = jnp.mean((z - mu) ** 2, axis=0)
        zn = (z - mu) * jax.lax.rsqrt(var + 1e-5) * g_ref[...] + beta_ref[...]
        out_ref[...] = jnp.maximum(zn, 0.0)

    return pl.pallas_call(
        body,
        out_shape=jax.ShapeDtypeStruct((n_dst, D), jnp.float32),
    )(acc, deg, h_prev, w_self, w_neigh, b, g, beta)


def _edge_mlp(huv, e_feat, wm1, bm1, wm2, bm2, n_pairs, edge_in, n_cls):
    def body(huv_ref, ef_ref, w1_ref, b1_ref, w2_ref, b2_ref, out_ref):
        hu = huv_ref[:n_pairs, :]
        hv = huv_ref[n_pairs:, :]
        t = (jnp.dot(hu, w1_ref[:D, :], preferred_element_type=jnp.float32)
             + jnp.dot(hv, w1_ref[D:2 * D, :],
                       preferred_element_type=jnp.float32)
             + jnp.dot(ef_ref[...], w1_ref[2 * D:, :],
                       preferred_element_type=jnp.float32)
             + b1_ref[...])
        t = jnp.maximum(t, 0.0)
        out_ref[...] = (jnp.dot(t, w2_ref[...],
                                preferred_element_type=jnp.float32)
                        + b2_ref[...])

    return pl.pallas_call(
        body,
        out_shape=jax.ShapeDtypeStruct((n_pairs, n_cls), jnp.float32),
    )(huv, e_feat, wm1, bm1, wm2, bm2)


def _pad_edges(src, dst, n_table, n_dst, n_dst_pad):
    e = src.shape[0]
    step = 2 * B * NW
    e_pad = ((e + step - 1) // step) * step
    if e_pad != e:
        pad = e_pad - e
        ar = jnp.arange(pad, dtype=jnp.int32)
        src = jnp.concatenate([src, ar % n_table])
        dst = jnp.concatenate([dst, n_dst + ar % (n_dst_pad - n_dst)])
    return src.reshape(e_pad // B, B), dst.reshape(e_pad // B, B), e_pad


def kernel(x_nodes, e_feat, W_self0, W_neigh0, b0, g0, beta0,
           W_self1, W_neigh1, b1, g1, beta1, Wm1, bm1, Wm2, bm2,
           edge_index0, edge_index1, pair_edges):
    n0 = x_nodes.shape[0]
    nd0 = 5000
    nd0p = 6144
    nd1 = 2048
    ep = pair_edges.shape[1]

    srcm0, dstm0, e0p = _pad_edges(edge_index0[0], edge_index0[1],
                                   n0, nd0, nd0p)
    acc0, deg0 = _make_segsum(n0, e0p, nd0p)(x_nodes, srcm0, dstm0)
    h0 = _sage_post(acc0, deg0, x_nodes, W_self0, W_neigh0, b0, g0, beta0,
                    nd0)

    srcm1, dstm1, e1p = _pad_edges(edge_index1[0], edge_index1[1],
                                   nd0, nd1, nd1)
    acc1, deg1 = _make_segsum(nd0, e1p, nd1)(h0, srcm1, dstm1)
    h1 = _sage_post(acc1, deg1, h0, W_self1, W_neigh1, b1, g1, beta1, nd1)

    uvm = jnp.concatenate([pair_edges[0], pair_edges[1]]).reshape(
        2 * ep // B, B)
    huv = _make_gather(nd1, 2 * ep)(h1, uvm)
    return _edge_mlp(huv, e_feat, Wm1, bm1, Wm2, bm2, ep,
                     e_feat.shape[1], Wm2.shape[1])
